# baseline (speedup 1.0000x reference)
import functools
import math

import jax
import jax.numpy as jnp
from jax import lax
from jax.experimental import pallas as pl
from jax.experimental.pallas import tpu as pltpu
from jax.experimental.pallas import tpu_sc as plsc

D_MODEL = 64
LANES = 16
NUM_CORES = 2
NUM_SUBCORES = 16
NUM_WORKERS = NUM_CORES * NUM_SUBCORES
CHUNK = 512
SCALE = math.sqrt(D_MODEL)


@functools.lru_cache(maxsize=None)
def _build(n_idx: int):
    assert n_idx % (NUM_WORKERS * CHUNK) == 0
    per_worker = n_idx // NUM_WORKERS
    n_chunks = per_worker // CHUNK
    col_groups = D_MODEL // LANES

    mesh = plsc.VectorSubcoreMesh(
        core_axis_name="c", subcore_axis_name="s",
        num_cores=NUM_CORES, num_subcores=NUM_SUBCORES)

    @functools.partial(
        pl.kernel,
        out_type=jax.ShapeDtypeStruct((n_idx, D_MODEL), jnp.float32),
        mesh=mesh,
        scratch_types=[
            pltpu.VMEM((CHUNK,), jnp.int32),
            pltpu.VMEM((CHUNK, D_MODEL), jnp.float32),
            pltpu.SemaphoreType.DMA,
        ],
        compiler_params=pltpu.CompilerParams(use_tc_tiling_on_sc=False),
    )
    def emb_kernel(table_hbm, idx_hbm, out_hbm, idx_v, rows_v, sem):
        wid = lax.axis_index("s") * NUM_CORES + lax.axis_index("c")
        base0 = wid * per_worker

        def chunk_body(ci, carry):
            base = base0 + ci * CHUNK
            pltpu.sync_copy(idx_hbm.at[pl.ds(base, CHUNK)], idx_v)
            pltpu.async_copy(table_hbm.at[idx_v], rows_v, sem).wait()

            def scale_row(r, c2):
                for j in range(col_groups):
                    rows_v[r, pl.ds(j * LANES, LANES)] = (
                        rows_v[r, pl.ds(j * LANES, LANES)] * SCALE)
                return c2

            lax.fori_loop(0, CHUNK, scale_row, 0, unroll=4)
            pltpu.sync_copy(rows_v, out_hbm.at[pl.ds(base, CHUNK)])
            return carry

        lax.fori_loop(0, n_chunks, chunk_body, 0)

    return emb_kernel


def kernel(x, table):
    n_idx = x.size
    flat_idx = x.reshape(n_idx)
    out = _build(n_idx)(table, flat_idx)
    return out.reshape(*x.shape, D_MODEL)
